# baseline (speedup 1.0000x reference)
import numpy as np
import jax
import jax.numpy as jnp
from jax.experimental import pallas as pl

_K = 5
_NEG = -3.0e38


def _lrelu(x, a):
    return jnp.where(x >= 0, x, a * x)


def _centers_row(n_rows):
    idx = jax.lax.broadcasted_iota(jnp.int32, (n_rows, 16), 1)
    return idx.astype(jnp.float32) * (5.0 / 15.0)


def _mol_core(xt, w1p, w2p, w3, aw2row, uwtp):
    n, _ = xt.shape
    xx = jnp.sum(xt * xt, axis=1, keepdims=True)
    s = jax.lax.dot_general(xt, xt, (((1,), (1,)), ((), ())),
                            preferred_element_type=jnp.float32,
                            precision=jax.lax.Precision.HIGHEST)
    pair = 2.0 * s - xx - jnp.transpose(xx)
    col = jax.lax.broadcasted_iota(jnp.int32, (n, n), 1)
    centw1 = jnp.dot(xt, w1p, preferred_element_type=jnp.float32, precision=jax.lax.Precision.HIGHEST)
    cent_row = _centers_row(n)

    work = pair
    neighs, logits = [], []
    for _ in range(_K):
        m = jnp.max(work, axis=1, keepdims=True)
        is_max = work >= m
        idxk = jnp.min(jnp.where(is_max, col, jnp.int32(2 ** 30)),
                       axis=1, keepdims=True)
        sel = col == idxk
        work = jnp.where(sel, _NEG, work)
        onehot = sel.astype(jnp.float32)
        neigh = jnp.dot(onehot, xt, preferred_element_type=jnp.float32, precision=jax.lax.Precision.HIGHEST)
        dist = jnp.sqrt(jnp.clip(-m, 1e-12, None))
        rbf = jnp.clip(jnp.exp(-10.0 * (dist - cent_row) ** 2), 1e-10, 1.0)
        h = _lrelu(centw1
                   + jnp.dot(neigh, w2p, preferred_element_type=jnp.float32, precision=jax.lax.Precision.HIGHEST)
                   + jnp.dot(rbf, w3, preferred_element_type=jnp.float32, precision=jax.lax.Precision.HIGHEST), 0.2)
        logits.append(jnp.sum(h * aw2row, axis=1, keepdims=True))
        neighs.append(neigh)

    mx = logits[0]
    for k in range(1, _K):
        mx = jnp.maximum(mx, logits[k])
    es = [jnp.exp(l - mx) for l in logits]
    z = es[0]
    for k in range(1, _K):
        z = z + es[k]

    upds = []
    for k in range(_K):
        att = es[k] / z
        nu = jnp.dot(neighs[k], uwtp, preferred_element_type=jnp.float32, precision=jax.lax.Precision.HIGHEST)
        upds.append(att * nu)
    mean_neigh = neighs[0]
    for k in range(1, _K):
        mean_neigh = mean_neigh + neighs[k]
    return upds, mean_neigh / float(_K)


def _bn_stats(s1, s2, count):
    mu = jnp.mean(s1, axis=0, keepdims=True) / count
    ex2 = jnp.mean(s2, axis=0, keepdims=True) / count
    var = ex2 - mu * mu
    rstd = 1.0 / jnp.sqrt(var + 1e-5)
    return mu, rstd


def _finalize_prev(upd_k, res_prev, mu, rstd, g, b):
    acc = None
    for k in range(_K):
        u = _lrelu((upd_k[k] - mu) * rstd * g + b, 0.02)
        acc = u if acc is None else acc + u
    return acc / float(_K) + 0.1 * res_prev


def _emit_layer(t, xt, wrefs, has_rw, upd_out, res_out, s1_out, s2_out, cout):
    w1p, w2p, w3, aw2, uwtp, rwtp = wrefs
    upds, mn = _mol_core(xt, w1p, w2p, w3, aw2, uwtp)
    ssum, ssq = None, None
    for k in range(_K):
        upd_out[t, k] = upds[k]
        cs = jnp.sum(upds[k], axis=0, keepdims=True)
        cq = jnp.sum(upds[k] * upds[k], axis=0, keepdims=True)
        ssum = cs if ssum is None else ssum + cs
        ssq = cq if ssq is None else ssq + cq
    res_out[t] = jnp.dot(mn, rwtp, preferred_element_type=jnp.float32, precision=jax.lax.Precision.HIGHEST) if has_rw else mn
    s1_out[...] += jnp.broadcast_to(ssum, (8, cout))
    s2_out[...] += jnp.broadcast_to(ssq, (8, cout))


def _layer_first_call(xt, w, tb, n, cout):
    b = xt.shape[0]
    w1p, w2p, w3, aw2, uwtp, rwtp = w
    cin = xt.shape[2]

    def body(x_ref, w1_ref, w2_ref, w3_ref, aw2_ref, uw_ref, rw_ref,
             upd_out, res_out, s1_out, s2_out):
        step = pl.program_id(0)

        @pl.when(step == 0)
        def _():
            s1_out[...] = jnp.zeros((8, cout), jnp.float32)
            s2_out[...] = jnp.zeros((8, cout), jnp.float32)

        wrefs = (w1_ref[...], w2_ref[...], w3_ref[...], aw2_ref[...],
                 uw_ref[...], rw_ref[...])
        for t in range(tb):
            _emit_layer(t, x_ref[t], wrefs, True, upd_out, res_out,
                        s1_out, s2_out, cout)

    grid = (b // tb,)
    const = lambda i: (0, 0)
    return pl.pallas_call(
        body,
        grid=grid,
        in_specs=[
            pl.BlockSpec((tb, n, cin), lambda i: (i, 0, 0)),
            pl.BlockSpec((cin, 64), const),
            pl.BlockSpec((cin, 64), const),
            pl.BlockSpec((16, 64), const),
            pl.BlockSpec((1, 64), const),
            pl.BlockSpec((cin, cout), const),
            pl.BlockSpec((cin, cout), const),
        ],
        out_specs=[
            pl.BlockSpec((tb, _K, n, cout), lambda i: (i, 0, 0, 0)),
            pl.BlockSpec((tb, n, cout), lambda i: (i, 0, 0)),
            pl.BlockSpec((8, cout), const),
            pl.BlockSpec((8, cout), const),
        ],
        out_shape=[
            jax.ShapeDtypeStruct((b, _K, n, cout), jnp.float32),
            jax.ShapeDtypeStruct((b, n, cout), jnp.float32),
            jax.ShapeDtypeStruct((8, cout), jnp.float32),
            jax.ShapeDtypeStruct((8, cout), jnp.float32),
        ],
    )(xt, *w)


def _layer_mid_call(prev, gp, bp, w, tb, n, cout, has_rw, bnk):
    upd_p, res_p, s1_p, s2_p = prev
    b = upd_p.shape[0]
    cp = upd_p.shape[3]
    w1p, w2p, w3, aw2, uwtp, rwtp = w
    cin = cp

    def body(updp_ref, resp_ref, s1p_ref, s2p_ref, gp_ref, bp_ref,
             w1_ref, w2_ref, w3_ref, aw2_ref, uw_ref, rw_ref,
             xc_out, upd_out, res_out, s1_out, s2_out):
        step = pl.program_id(0)

        @pl.when(step == 0)
        def _():
            s1_out[...] = jnp.zeros((8, cout), jnp.float32)
            s2_out[...] = jnp.zeros((8, cout), jnp.float32)

        mu, rstd = _bn_stats(s1p_ref[...], s2p_ref[...], float(bnk))
        gpv, bpv = gp_ref[...], bp_ref[...]
        wrefs = (w1_ref[...], w2_ref[...], w3_ref[...], aw2_ref[...],
                 uw_ref[...], rw_ref[...] if rw_ref is not None else None)
        for t in range(tb):
            upd_k = [updp_ref[t, k] for k in range(_K)]
            cur = _finalize_prev(upd_k, resp_ref[t], mu, rstd, gpv, bpv)
            xc_out[t] = cur
            _emit_layer(t, cur, wrefs, has_rw, upd_out, res_out,
                        s1_out, s2_out, cout)

    grid = (b // tb,)
    const = lambda i: (0, 0)
    in_specs = [
        pl.BlockSpec((tb, _K, n, cp), lambda i: (i, 0, 0, 0)),
        pl.BlockSpec((tb, n, cp), lambda i: (i, 0, 0)),
        pl.BlockSpec((8, cp), const),
        pl.BlockSpec((8, cp), const),
        pl.BlockSpec((1, cp), const),
        pl.BlockSpec((1, cp), const),
        pl.BlockSpec((cin, 64), const),
        pl.BlockSpec((cin, 64), const),
        pl.BlockSpec((16, 64), const),
        pl.BlockSpec((1, 64), const),
        pl.BlockSpec((cin, cout), const),
    ]
    args = [upd_p, res_p, s1_p, s2_p, gp, bp, w1p, w2p, w3, aw2, uwtp]
    if has_rw:
        in_specs.append(pl.BlockSpec((cin, cout), const))
        args.append(rwtp)
        fn = body
    else:
        def fn(updp_ref, resp_ref, s1p_ref, s2p_ref, gp_ref, bp_ref,
               w1_ref, w2_ref, w3_ref, aw2_ref, uw_ref,
               xc_out, upd_out, res_out, s1_out, s2_out):
            body(updp_ref, resp_ref, s1p_ref, s2p_ref, gp_ref, bp_ref,
                 w1_ref, w2_ref, w3_ref, aw2_ref, uw_ref, None,
                 xc_out, upd_out, res_out, s1_out, s2_out)

    return pl.pallas_call(
        fn,
        grid=grid,
        in_specs=in_specs,
        out_specs=[
            pl.BlockSpec((tb, n, cp), lambda i: (i, 0, 0)),
            pl.BlockSpec((tb, _K, n, cout), lambda i: (i, 0, 0, 0)),
            pl.BlockSpec((tb, n, cout), lambda i: (i, 0, 0)),
            pl.BlockSpec((8, cout), const),
            pl.BlockSpec((8, cout), const),
        ],
        out_shape=[
            jax.ShapeDtypeStruct((b, n, cp), jnp.float32),
            jax.ShapeDtypeStruct((b, _K, n, cout), jnp.float32),
            jax.ShapeDtypeStruct((b, n, cout), jnp.float32),
            jax.ShapeDtypeStruct((8, cout), jnp.float32),
            jax.ShapeDtypeStruct((8, cout), jnp.float32),
        ],
    )(*args)


def _conv_call(xcs, prev, gp, bp, wcs, tb, n, emb, bnk):
    xc0, xc1, xc2 = xcs
    upd_p, res_p, s1_p, s2_p = prev
    b = upd_p.shape[0]
    cp = upd_p.shape[3]
    c0, c1, c2 = xc0.shape[2], xc1.shape[2], xc2.shape[2]
    wc0, wc1, wc2, wc3 = wcs

    def body(xc0_ref, xc1_ref, xc2_ref, updp_ref, resp_ref, s1p_ref, s2p_ref,
             gp_ref, bp_ref, wc0_ref, wc1_ref, wc2_ref, wc3_ref,
             y_out, ys1_out, ys2_out):
        step = pl.program_id(0)

        @pl.when(step == 0)
        def _():
            ys1_out[...] = jnp.zeros((8, emb), jnp.float32)
            ys2_out[...] = jnp.zeros((8, emb), jnp.float32)

        mu, rstd = _bn_stats(s1p_ref[...], s2p_ref[...], float(bnk))
        gpv, bpv = gp_ref[...], bp_ref[...]
        for t in range(tb):
            upd_k = [updp_ref[t, k] for k in range(_K)]
            cur3 = _finalize_prev(upd_k, resp_ref[t], mu, rstd, gpv, bpv)
            y = (jnp.dot(xc0_ref[t], wc0_ref[...], preferred_element_type=jnp.float32, precision=jax.lax.Precision.HIGHEST)
                 + jnp.dot(xc1_ref[t], wc1_ref[...], preferred_element_type=jnp.float32, precision=jax.lax.Precision.HIGHEST)
                 + jnp.dot(xc2_ref[t], wc2_ref[...], preferred_element_type=jnp.float32, precision=jax.lax.Precision.HIGHEST)
                 + jnp.dot(cur3, wc3_ref[...], preferred_element_type=jnp.float32, precision=jax.lax.Precision.HIGHEST))
            y_out[t] = y
            ys1_out[...] += jnp.broadcast_to(
                jnp.sum(y, axis=0, keepdims=True), (8, emb))
            ys2_out[...] += jnp.broadcast_to(
                jnp.sum(y * y, axis=0, keepdims=True), (8, emb))

    grid = (b // tb,)
    const = lambda i: (0, 0)
    return pl.pallas_call(
        body,
        grid=grid,
        in_specs=[
            pl.BlockSpec((tb, n, c0), lambda i: (i, 0, 0)),
            pl.BlockSpec((tb, n, c1), lambda i: (i, 0, 0)),
            pl.BlockSpec((tb, n, c2), lambda i: (i, 0, 0)),
            pl.BlockSpec((tb, _K, n, cp), lambda i: (i, 0, 0, 0)),
            pl.BlockSpec((tb, n, cp), lambda i: (i, 0, 0)),
            pl.BlockSpec((8, cp), const),
            pl.BlockSpec((8, cp), const),
            pl.BlockSpec((1, cp), const),
            pl.BlockSpec((1, cp), const),
            pl.BlockSpec((c0, emb), const),
            pl.BlockSpec((c1, emb), const),
            pl.BlockSpec((c2, emb), const),
            pl.BlockSpec((cp, emb), const),
        ],
        out_specs=[
            pl.BlockSpec((tb, n, emb), lambda i: (i, 0, 0)),
            pl.BlockSpec((8, emb), const),
            pl.BlockSpec((8, emb), const),
        ],
        out_shape=[
            jax.ShapeDtypeStruct((b, n, emb), jnp.float32),
            jax.ShapeDtypeStruct((8, emb), jnp.float32),
            jax.ShapeDtypeStruct((8, emb), jnp.float32),
        ],
    )(xc0, xc1, xc2, upd_p, res_p, s1_p, s2_p, gp, bp, wc0, wc1, wc2, wc3)


def _pool_call(y, ys1, ys2, g, bb, tb, n, emb, bn_count):
    b = y.shape[0]

    def body(y_ref, s1_ref, s2_ref, g_ref, b_ref, p1_out, p2_out):
        mu, rstd = _bn_stats(s1_ref[...], s2_ref[...], float(bn_count))
        gv, bv = g_ref[...], b_ref[...]
        for t in range(tb):
            z = _lrelu((y_ref[t] - mu) * rstd * gv + bv, 0.2)
            p1_out[pl.ds(t, 1), :] = jnp.max(z, axis=0, keepdims=True)
            p2_out[pl.ds(t, 1), :] = jnp.mean(z, axis=0, keepdims=True)

    grid = (b // tb,)
    const = lambda i: (0, 0)
    return pl.pallas_call(
        body,
        grid=grid,
        in_specs=[
            pl.BlockSpec((tb, n, emb), lambda i: (i, 0, 0)),
            pl.BlockSpec((8, emb), const),
            pl.BlockSpec((8, emb), const),
            pl.BlockSpec((1, emb), const),
            pl.BlockSpec((1, emb), const),
        ],
        out_specs=[
            pl.BlockSpec((tb, emb), lambda i: (i, 0)),
            pl.BlockSpec((tb, emb), lambda i: (i, 0)),
        ],
        out_shape=[
            jax.ShapeDtypeStruct((b, emb), jnp.float32),
            jax.ShapeDtypeStruct((b, emb), jnp.float32),
        ],
    )(y, ys1, ys2, g, bb)


def _ln_rows(v):
    mu = jnp.mean(v, axis=1, keepdims=True)
    var = jnp.mean((v - mu) ** 2, axis=1, keepdims=True)
    return (v - mu) / jnp.sqrt(var + 1e-5)


def _head_call(p1, p2, env2, wm1, wm2, gm, bm,
               d0w1a, d0w1b, d0w2t, d0w3t, p0m, p0e,
               d1w1t, d1w2t, d1w3t, d2w1t, d2w2t, d2w3t, p2map,
               fct, fcb, out_dim):
    b = p1.shape[0]

    def body(p1_ref, p2_ref, e_ref, wm1_ref, wm2_ref, gm_ref, bm_ref,
             a_ref, b1_ref, w02_ref, w03_ref, p0m_ref, p0e_ref,
             w11_ref, w12_ref, w13_ref, w21_ref, w22_ref, w23_ref, p2m_ref,
             fct_ref, fcb_ref, out_ref):
        m0 = (jnp.dot(p1_ref[...], wm1_ref[...], preferred_element_type=jnp.float32, precision=jax.lax.Precision.HIGHEST)
              + jnp.dot(p2_ref[...], wm2_ref[...], preferred_element_type=jnp.float32, precision=jax.lax.Precision.HIGHEST))
        mu = jnp.mean(m0, axis=0, keepdims=True)
        var = jnp.mean((m0 - mu) ** 2, axis=0, keepdims=True)
        m = _lrelu((m0 - mu) / jnp.sqrt(var + 1e-5) * gm_ref[...] + bm_ref[...], 0.2)
        e = e_ref[...]

        t = _lrelu(_ln_rows(jnp.dot(m, a_ref[...], preferred_element_type=jnp.float32, precision=jax.lax.Precision.HIGHEST)
                            + e * b1_ref[...]), 0.2)
        t = _lrelu(_ln_rows(jnp.dot(t, w02_ref[...], preferred_element_type=jnp.float32, precision=jax.lax.Precision.HIGHEST)), 0.2)
        t = _ln_rows(jnp.dot(t, w03_ref[...], preferred_element_type=jnp.float32, precision=jax.lax.Precision.HIGHEST))
        idn = jnp.dot(m, p0m_ref[...], preferred_element_type=jnp.float32, precision=jax.lax.Precision.HIGHEST) + e * p0e_ref[...]
        h1 = _lrelu(t + idn, 0.2)

        t = _lrelu(_ln_rows(jnp.dot(h1, w11_ref[...], preferred_element_type=jnp.float32, precision=jax.lax.Precision.HIGHEST)), 0.2)
        t = _lrelu(_ln_rows(jnp.dot(t, w12_ref[...], preferred_element_type=jnp.float32, precision=jax.lax.Precision.HIGHEST)), 0.2)
        t = _ln_rows(jnp.dot(t, w13_ref[...], preferred_element_type=jnp.float32, precision=jax.lax.Precision.HIGHEST))
        h2 = _lrelu(t + h1, 0.2)

        t = _lrelu(_ln_rows(jnp.dot(h2, w21_ref[...], preferred_element_type=jnp.float32, precision=jax.lax.Precision.HIGHEST)), 0.2)
        t = _lrelu(_ln_rows(jnp.dot(t, w22_ref[...], preferred_element_type=jnp.float32, precision=jax.lax.Precision.HIGHEST)), 0.2)
        t = _ln_rows(jnp.dot(t, w23_ref[...], preferred_element_type=jnp.float32, precision=jax.lax.Precision.HIGHEST))
        h3 = _lrelu(t + jnp.dot(h2, p2m_ref[...], preferred_element_type=jnp.float32, precision=jax.lax.Precision.HIGHEST), 0.2)

        out_ref[...] = (jnp.dot(h3, fct_ref[...], preferred_element_type=jnp.float32, precision=jax.lax.Precision.HIGHEST)
                        + fcb_ref[...])

    return pl.pallas_call(
        body,
        out_shape=jax.ShapeDtypeStruct((b, out_dim), jnp.float32),
    )(p1, p2, env2, wm1, wm2, gm, bm, d0w1a, d0w1b, d0w2t, d0w3t, p0m, p0e,
      d1w1t, d1w2t, d1w3t, d2w1t, d2w2t, d2w3t, p2map, fct, fcb)


def _prep_layer_weights(aw1, aw2, uw, rw, cin, rm):
    eff = cin - 3 if rm else cin
    w1 = jnp.transpose(aw1[:, :eff])
    w2 = jnp.transpose(aw1[:, eff:2 * eff])
    w3 = jnp.transpose(aw1[:, 2 * eff:])
    uwt = jnp.transpose(uw)
    rwt = jnp.transpose(rw) if rw is not None else None
    if rm:
        pad = jnp.zeros((3, 64), jnp.float32)
        w1 = jnp.concatenate([pad, w1], axis=0)
        w2 = jnp.concatenate([pad, w2], axis=0)
        padc = jnp.zeros((3, uwt.shape[1]), jnp.float32)
        uwt = jnp.concatenate([padc, uwt], axis=0)
        if rwt is not None:
            rwt = jnp.concatenate([jnp.zeros((3, rwt.shape[1]), jnp.float32),
                                   rwt], axis=0)
    return w1, w2, w3, aw2, uwt, rwt


def kernel(x, env, idx_base, mc0_aw1, mc0_aw2, mc0_uw, mc0_bg, mc0_bb, mc0_rw,
           mc1_aw1, mc1_aw2, mc1_uw, mc1_bg, mc1_bb,
           mc2_aw1, mc2_aw2, mc2_uw, mc2_bg, mc2_bb, mc2_rw,
           mc3_aw1, mc3_aw2, mc3_uw, mc3_bg, mc3_bb,
           conv_w, conv_bg, conv_bb, mrg_w, mrg_bg, mrg_bb,
           dec0_w1, dec0_w2, dec0_w3, dec1_w1, dec1_w2, dec1_w3,
           dec2_w1, dec2_w2, dec2_w3, fc_w, fc_b):
    b, cin0, n = x.shape
    emb = conv_w.shape[0]
    out_dim = fc_w.shape[0]
    tb = 4 if b % 4 == 0 else 1
    tb6 = 8 if b % 8 == 0 else 1
    bnk = b * n * _K

    xt = jnp.transpose(x, (0, 2, 1))

    w0 = _prep_layer_weights(mc0_aw1, mc0_aw2, mc0_uw, mc0_rw, cin0, True)
    w1 = _prep_layer_weights(mc1_aw1, mc1_aw2, mc1_uw, None, 32, False)
    w2 = _prep_layer_weights(mc2_aw1, mc2_aw2, mc2_uw, mc2_rw, 32, False)
    w3 = _prep_layer_weights(mc3_aw1, mc3_aw2, mc3_uw, None, 64, False)

    r2 = lambda v: v.reshape(1, -1)

    p0 = _layer_first_call(xt, w0, tb, n, 32)
    upd0, res0, s10, s20 = p0
    xc0, upd1, res1, s11, s21 = _layer_mid_call(
        (upd0, res0, s10, s20), r2(mc0_bg), r2(mc0_bb), w1, tb, n, 32,
        False, bnk)
    xc1, upd2, res2, s12, s22 = _layer_mid_call(
        (upd1, res1, s11, s21), r2(mc1_bg), r2(mc1_bb), w2, tb, n, 64,
        True, bnk)
    xc2, upd3, res3, s13, s23 = _layer_mid_call(
        (upd2, res2, s12, s22), r2(mc2_bg), r2(mc2_bb), w3, tb, n, 64,
        False, bnk)

    cw = jnp.transpose(conv_w)
    wcs = (cw[0:32], cw[32:64], cw[64:128], cw[128:192])
    y, ys1, ys2 = _conv_call((xc0, xc1, xc2), (upd3, res3, s13, s23),
                             r2(mc3_bg), r2(mc3_bb), wcs, tb, n, emb, bnk)

    p1, p2 = _pool_call(y, ys1, ys2, r2(conv_bg), r2(conv_bb), tb6, n, emb,
                        b * n)

    mrg_t = jnp.transpose(mrg_w)
    wm1, wm2 = mrg_t[:emb], mrg_t[emb:]
    d0w1t = jnp.transpose(dec0_w1)
    d0w1a, d0w1b = d0w1t[:emb], d0w1t[emb:]
    di0, do0 = dec0_w1.shape[1], dec0_w1.shape[0]
    p0full = (np.arange(di0)[:, None]
              == (np.arange(do0)[None, :] * di0) // do0).astype(np.float32)
    p0m = jnp.asarray(p0full[:emb])
    p0e = jnp.asarray(p0full[emb:])
    di2, do2 = dec2_w1.shape[1], dec2_w1.shape[0]
    p2map = jnp.asarray((np.arange(di2)[:, None]
                         == (np.arange(do2)[None, :] * di2) // do2)
                        .astype(np.float32))

    return _head_call(
        p1, p2, env.reshape(-1, 1), wm1, wm2, r2(mrg_bg), r2(mrg_bb),
        d0w1a, d0w1b, jnp.transpose(dec0_w2), jnp.transpose(dec0_w3),
        p0m, p0e,
        jnp.transpose(dec1_w1), jnp.transpose(dec1_w2), jnp.transpose(dec1_w3),
        jnp.transpose(dec2_w1), jnp.transpose(dec2_w2), jnp.transpose(dec2_w3),
        p2map,
        jnp.transpose(fc_w), fc_b.reshape(1, -1), out_dim)


# speedup vs baseline: 1.0129x; 1.0129x over previous
import numpy as np
import jax
import jax.numpy as jnp
from jax.experimental import pallas as pl

_K = 5
_NEG = -3.0e38


def _lrelu(x, a):
    return jnp.where(x >= 0, x, a * x)


def _centers_row(n_rows):
    idx = jax.lax.broadcasted_iota(jnp.int32, (n_rows, 16), 1)
    return idx.astype(jnp.float32) * (5.0 / 15.0)


def _mol_core(xt, w1p, w2p, w3, aw2row, uwtp):
    n, _ = xt.shape
    xx = jnp.sum(xt * xt, axis=1, keepdims=True)
    s = jax.lax.dot_general(xt, xt, (((1,), (1,)), ((), ())),
                            preferred_element_type=jnp.float32,
                            precision=jax.lax.Precision.HIGHEST)
    pair = 2.0 * s - xx - jnp.transpose(xx)
    row = jax.lax.broadcasted_iota(jnp.int32, (n, n), 0)
    centw1 = jnp.dot(xt, w1p, preferred_element_type=jnp.float32, precision=jax.lax.Precision.HIGHEST)
    cent_row = _centers_row(n)

    work = pair
    neighs, logits = [], []
    for _ in range(_K):
        m = jnp.max(work, axis=0, keepdims=True)
        is_max = work >= m
        idxk = jnp.min(jnp.where(is_max, row, jnp.int32(2 ** 30)),
                       axis=0, keepdims=True)
        sel = row == idxk
        work = jnp.where(sel, _NEG, work)
        onehot = sel.astype(jnp.float32)
        neigh = jax.lax.dot_general(onehot, xt, (((0,), (0,)), ((), ())),
                                    preferred_element_type=jnp.float32,
                                    precision=jax.lax.Precision.HIGHEST)
        dist = jnp.sqrt(jnp.clip(-jnp.transpose(m), 1e-12, None))
        rbf = jnp.clip(jnp.exp(-10.0 * (dist - cent_row) ** 2), 1e-10, 1.0)
        h = _lrelu(centw1
                   + jnp.dot(neigh, w2p, preferred_element_type=jnp.float32, precision=jax.lax.Precision.HIGHEST)
                   + jnp.dot(rbf, w3, preferred_element_type=jnp.float32, precision=jax.lax.Precision.HIGHEST), 0.2)
        logits.append(jnp.sum(h * aw2row, axis=1, keepdims=True))
        neighs.append(neigh)

    mx = logits[0]
    for k in range(1, _K):
        mx = jnp.maximum(mx, logits[k])
    es = [jnp.exp(l - mx) for l in logits]
    z = es[0]
    for k in range(1, _K):
        z = z + es[k]

    upds = []
    for k in range(_K):
        att = es[k] / z
        nu = jnp.dot(neighs[k], uwtp, preferred_element_type=jnp.float32, precision=jax.lax.Precision.HIGHEST)
        upds.append(att * nu)
    mean_neigh = neighs[0]
    for k in range(1, _K):
        mean_neigh = mean_neigh + neighs[k]
    return upds, mean_neigh / float(_K)


def _bn_stats(s1, s2, count):
    mu = jnp.mean(s1, axis=0, keepdims=True) / count
    ex2 = jnp.mean(s2, axis=0, keepdims=True) / count
    var = ex2 - mu * mu
    rstd = 1.0 / jnp.sqrt(var + 1e-5)
    return mu, rstd


def _finalize_prev(upd_k, res_prev, mu, rstd, g, b):
    acc = None
    for k in range(_K):
        u = _lrelu((upd_k[k] - mu) * rstd * g + b, 0.02)
        acc = u if acc is None else acc + u
    return acc / float(_K) + 0.1 * res_prev


def _emit_layer(t, xt, wrefs, has_rw, upd_out, res_out, s1_out, s2_out, cout):
    w1p, w2p, w3, aw2, uwtp, rwtp = wrefs
    upds, mn = _mol_core(xt, w1p, w2p, w3, aw2, uwtp)
    ssum, ssq = None, None
    for k in range(_K):
        upd_out[t, k] = upds[k]
        cs = jnp.sum(upds[k], axis=0, keepdims=True)
        cq = jnp.sum(upds[k] * upds[k], axis=0, keepdims=True)
        ssum = cs if ssum is None else ssum + cs
        ssq = cq if ssq is None else ssq + cq
    res_out[t] = jnp.dot(mn, rwtp, preferred_element_type=jnp.float32, precision=jax.lax.Precision.HIGHEST) if has_rw else mn
    s1_out[...] += jnp.broadcast_to(ssum, (8, cout))
    s2_out[...] += jnp.broadcast_to(ssq, (8, cout))


def _layer_first_call(xt, w, tb, n, cout):
    b = xt.shape[0]
    w1p, w2p, w3, aw2, uwtp, rwtp = w
    cin = xt.shape[2]

    def body(x_ref, w1_ref, w2_ref, w3_ref, aw2_ref, uw_ref, rw_ref,
             upd_out, res_out, s1_out, s2_out):
        step = pl.program_id(0)

        @pl.when(step == 0)
        def _():
            s1_out[...] = jnp.zeros((8, cout), jnp.float32)
            s2_out[...] = jnp.zeros((8, cout), jnp.float32)

        wrefs = (w1_ref[...], w2_ref[...], w3_ref[...], aw2_ref[...],
                 uw_ref[...], rw_ref[...])
        for t in range(tb):
            _emit_layer(t, x_ref[t], wrefs, True, upd_out, res_out,
                        s1_out, s2_out, cout)

    grid = (b // tb,)
    const = lambda i: (0, 0)
    return pl.pallas_call(
        body,
        grid=grid,
        in_specs=[
            pl.BlockSpec((tb, n, cin), lambda i: (i, 0, 0)),
            pl.BlockSpec((cin, 64), const),
            pl.BlockSpec((cin, 64), const),
            pl.BlockSpec((16, 64), const),
            pl.BlockSpec((1, 64), const),
            pl.BlockSpec((cin, cout), const),
            pl.BlockSpec((cin, cout), const),
        ],
        out_specs=[
            pl.BlockSpec((tb, _K, n, cout), lambda i: (i, 0, 0, 0)),
            pl.BlockSpec((tb, n, cout), lambda i: (i, 0, 0)),
            pl.BlockSpec((8, cout), const),
            pl.BlockSpec((8, cout), const),
        ],
        out_shape=[
            jax.ShapeDtypeStruct((b, _K, n, cout), jnp.float32),
            jax.ShapeDtypeStruct((b, n, cout), jnp.float32),
            jax.ShapeDtypeStruct((8, cout), jnp.float32),
            jax.ShapeDtypeStruct((8, cout), jnp.float32),
        ],
    )(xt, *w)


def _layer_mid_call(prev, gp, bp, w, tb, n, cout, has_rw, bnk):
    upd_p, res_p, s1_p, s2_p = prev
    b = upd_p.shape[0]
    cp = upd_p.shape[3]
    w1p, w2p, w3, aw2, uwtp, rwtp = w
    cin = cp

    def body(updp_ref, resp_ref, s1p_ref, s2p_ref, gp_ref, bp_ref,
             w1_ref, w2_ref, w3_ref, aw2_ref, uw_ref, rw_ref,
             xc_out, upd_out, res_out, s1_out, s2_out):
        step = pl.program_id(0)

        @pl.when(step == 0)
        def _():
            s1_out[...] = jnp.zeros((8, cout), jnp.float32)
            s2_out[...] = jnp.zeros((8, cout), jnp.float32)

        mu, rstd = _bn_stats(s1p_ref[...], s2p_ref[...], float(bnk))
        gpv, bpv = gp_ref[...], bp_ref[...]
        wrefs = (w1_ref[...], w2_ref[...], w3_ref[...], aw2_ref[...],
                 uw_ref[...], rw_ref[...] if rw_ref is not None else None)
        for t in range(tb):
            upd_k = [updp_ref[t, k] for k in range(_K)]
            cur = _finalize_prev(upd_k, resp_ref[t], mu, rstd, gpv, bpv)
            xc_out[t] = cur
            _emit_layer(t, cur, wrefs, has_rw, upd_out, res_out,
                        s1_out, s2_out, cout)

    grid = (b // tb,)
    const = lambda i: (0, 0)
    in_specs = [
        pl.BlockSpec((tb, _K, n, cp), lambda i: (i, 0, 0, 0)),
        pl.BlockSpec((tb, n, cp), lambda i: (i, 0, 0)),
        pl.BlockSpec((8, cp), const),
        pl.BlockSpec((8, cp), const),
        pl.BlockSpec((1, cp), const),
        pl.BlockSpec((1, cp), const),
        pl.BlockSpec((cin, 64), const),
        pl.BlockSpec((cin, 64), const),
        pl.BlockSpec((16, 64), const),
        pl.BlockSpec((1, 64), const),
        pl.BlockSpec((cin, cout), const),
    ]
    args = [upd_p, res_p, s1_p, s2_p, gp, bp, w1p, w2p, w3, aw2, uwtp]
    if has_rw:
        in_specs.append(pl.BlockSpec((cin, cout), const))
        args.append(rwtp)
        fn = body
    else:
        def fn(updp_ref, resp_ref, s1p_ref, s2p_ref, gp_ref, bp_ref,
               w1_ref, w2_ref, w3_ref, aw2_ref, uw_ref,
               xc_out, upd_out, res_out, s1_out, s2_out):
            body(updp_ref, resp_ref, s1p_ref, s2p_ref, gp_ref, bp_ref,
                 w1_ref, w2_ref, w3_ref, aw2_ref, uw_ref, None,
                 xc_out, upd_out, res_out, s1_out, s2_out)

    return pl.pallas_call(
        fn,
        grid=grid,
        in_specs=in_specs,
        out_specs=[
            pl.BlockSpec((tb, n, cp), lambda i: (i, 0, 0)),
            pl.BlockSpec((tb, _K, n, cout), lambda i: (i, 0, 0, 0)),
            pl.BlockSpec((tb, n, cout), lambda i: (i, 0, 0)),
            pl.BlockSpec((8, cout), const),
            pl.BlockSpec((8, cout), const),
        ],
        out_shape=[
            jax.ShapeDtypeStruct((b, n, cp), jnp.float32),
            jax.ShapeDtypeStruct((b, _K, n, cout), jnp.float32),
            jax.ShapeDtypeStruct((b, n, cout), jnp.float32),
            jax.ShapeDtypeStruct((8, cout), jnp.float32),
            jax.ShapeDtypeStruct((8, cout), jnp.float32),
        ],
    )(*args)


def _conv_call(xcs, prev, gp, bp, wcs, tb, n, emb, bnk):
    xc0, xc1, xc2 = xcs
    upd_p, res_p, s1_p, s2_p = prev
    b = upd_p.shape[0]
    cp = upd_p.shape[3]
    c0, c1, c2 = xc0.shape[2], xc1.shape[2], xc2.shape[2]
    wc0, wc1, wc2, wc3 = wcs

    def body(xc0_ref, xc1_ref, xc2_ref, updp_ref, resp_ref, s1p_ref, s2p_ref,
             gp_ref, bp_ref, wc0_ref, wc1_ref, wc2_ref, wc3_ref,
             y_out, ys1_out, ys2_out):
        step = pl.program_id(0)

        @pl.when(step == 0)
        def _():
            ys1_out[...] = jnp.zeros((8, emb), jnp.float32)
            ys2_out[...] = jnp.zeros((8, emb), jnp.float32)

        mu, rstd = _bn_stats(s1p_ref[...], s2p_ref[...], float(bnk))
        gpv, bpv = gp_ref[...], bp_ref[...]
        for t in range(tb):
            upd_k = [updp_ref[t, k] for k in range(_K)]
            cur3 = _finalize_prev(upd_k, resp_ref[t], mu, rstd, gpv, bpv)
            y = (jnp.dot(xc0_ref[t], wc0_ref[...], preferred_element_type=jnp.float32, precision=jax.lax.Precision.HIGHEST)
                 + jnp.dot(xc1_ref[t], wc1_ref[...], preferred_element_type=jnp.float32, precision=jax.lax.Precision.HIGHEST)
                 + jnp.dot(xc2_ref[t], wc2_ref[...], preferred_element_type=jnp.float32, precision=jax.lax.Precision.HIGHEST)
                 + jnp.dot(cur3, wc3_ref[...], preferred_element_type=jnp.float32, precision=jax.lax.Precision.HIGHEST))
            y_out[t] = y
            ys1_out[...] += jnp.broadcast_to(
                jnp.sum(y, axis=0, keepdims=True), (8, emb))
            ys2_out[...] += jnp.broadcast_to(
                jnp.sum(y * y, axis=0, keepdims=True), (8, emb))

    grid = (b // tb,)
    const = lambda i: (0, 0)
    return pl.pallas_call(
        body,
        grid=grid,
        in_specs=[
            pl.BlockSpec((tb, n, c0), lambda i: (i, 0, 0)),
            pl.BlockSpec((tb, n, c1), lambda i: (i, 0, 0)),
            pl.BlockSpec((tb, n, c2), lambda i: (i, 0, 0)),
            pl.BlockSpec((tb, _K, n, cp), lambda i: (i, 0, 0, 0)),
            pl.BlockSpec((tb, n, cp), lambda i: (i, 0, 0)),
            pl.BlockSpec((8, cp), const),
            pl.BlockSpec((8, cp), const),
            pl.BlockSpec((1, cp), const),
            pl.BlockSpec((1, cp), const),
            pl.BlockSpec((c0, emb), const),
            pl.BlockSpec((c1, emb), const),
            pl.BlockSpec((c2, emb), const),
            pl.BlockSpec((cp, emb), const),
        ],
        out_specs=[
            pl.BlockSpec((tb, n, emb), lambda i: (i, 0, 0)),
            pl.BlockSpec((8, emb), const),
            pl.BlockSpec((8, emb), const),
        ],
        out_shape=[
            jax.ShapeDtypeStruct((b, n, emb), jnp.float32),
            jax.ShapeDtypeStruct((8, emb), jnp.float32),
            jax.ShapeDtypeStruct((8, emb), jnp.float32),
        ],
    )(xc0, xc1, xc2, upd_p, res_p, s1_p, s2_p, gp, bp, wc0, wc1, wc2, wc3)


def _pool_call(y, ys1, ys2, g, bb, tb, n, emb, bn_count):
    b = y.shape[0]

    def body(y_ref, s1_ref, s2_ref, g_ref, b_ref, p1_out, p2_out):
        mu, rstd = _bn_stats(s1_ref[...], s2_ref[...], float(bn_count))
        gv, bv = g_ref[...], b_ref[...]
        for t in range(tb):
            z = _lrelu((y_ref[t] - mu) * rstd * gv + bv, 0.2)
            p1_out[pl.ds(t, 1), :] = jnp.max(z, axis=0, keepdims=True)
            p2_out[pl.ds(t, 1), :] = jnp.mean(z, axis=0, keepdims=True)

    grid = (b // tb,)
    const = lambda i: (0, 0)
    return pl.pallas_call(
        body,
        grid=grid,
        in_specs=[
            pl.BlockSpec((tb, n, emb), lambda i: (i, 0, 0)),
            pl.BlockSpec((8, emb), const),
            pl.BlockSpec((8, emb), const),
            pl.BlockSpec((1, emb), const),
            pl.BlockSpec((1, emb), const),
        ],
        out_specs=[
            pl.BlockSpec((tb, emb), lambda i: (i, 0)),
            pl.BlockSpec((tb, emb), lambda i: (i, 0)),
        ],
        out_shape=[
            jax.ShapeDtypeStruct((b, emb), jnp.float32),
            jax.ShapeDtypeStruct((b, emb), jnp.float32),
        ],
    )(y, ys1, ys2, g, bb)


def _ln_rows(v):
    mu = jnp.mean(v, axis=1, keepdims=True)
    var = jnp.mean((v - mu) ** 2, axis=1, keepdims=True)
    return (v - mu) / jnp.sqrt(var + 1e-5)


def _head_call(p1, p2, env2, wm1, wm2, gm, bm,
               d0w1a, d0w1b, d0w2t, d0w3t, p0m, p0e,
               d1w1t, d1w2t, d1w3t, d2w1t, d2w2t, d2w3t, p2map,
               fct, fcb, out_dim):
    b = p1.shape[0]

    def body(p1_ref, p2_ref, e_ref, wm1_ref, wm2_ref, gm_ref, bm_ref,
             a_ref, b1_ref, w02_ref, w03_ref, p0m_ref, p0e_ref,
             w11_ref, w12_ref, w13_ref, w21_ref, w22_ref, w23_ref, p2m_ref,
             fct_ref, fcb_ref, out_ref):
        m0 = (jnp.dot(p1_ref[...], wm1_ref[...], preferred_element_type=jnp.float32, precision=jax.lax.Precision.HIGHEST)
              + jnp.dot(p2_ref[...], wm2_ref[...], preferred_element_type=jnp.float32, precision=jax.lax.Precision.HIGHEST))
        mu = jnp.mean(m0, axis=0, keepdims=True)
        var = jnp.mean((m0 - mu) ** 2, axis=0, keepdims=True)
        m = _lrelu((m0 - mu) / jnp.sqrt(var + 1e-5) * gm_ref[...] + bm_ref[...], 0.2)
        e = e_ref[...]

        t = _lrelu(_ln_rows(jnp.dot(m, a_ref[...], preferred_element_type=jnp.float32, precision=jax.lax.Precision.HIGHEST)
                            + e * b1_ref[...]), 0.2)
        t = _lrelu(_ln_rows(jnp.dot(t, w02_ref[...], preferred_element_type=jnp.float32, precision=jax.lax.Precision.HIGHEST)), 0.2)
        t = _ln_rows(jnp.dot(t, w03_ref[...], preferred_element_type=jnp.float32, precision=jax.lax.Precision.HIGHEST))
        idn = jnp.dot(m, p0m_ref[...], preferred_element_type=jnp.float32, precision=jax.lax.Precision.HIGHEST) + e * p0e_ref[...]
        h1 = _lrelu(t + idn, 0.2)

        t = _lrelu(_ln_rows(jnp.dot(h1, w11_ref[...], preferred_element_type=jnp.float32, precision=jax.lax.Precision.HIGHEST)), 0.2)
        t = _lrelu(_ln_rows(jnp.dot(t, w12_ref[...], preferred_element_type=jnp.float32, precision=jax.lax.Precision.HIGHEST)), 0.2)
        t = _ln_rows(jnp.dot(t, w13_ref[...], preferred_element_type=jnp.float32, precision=jax.lax.Precision.HIGHEST))
        h2 = _lrelu(t + h1, 0.2)

        t = _lrelu(_ln_rows(jnp.dot(h2, w21_ref[...], preferred_element_type=jnp.float32, precision=jax.lax.Precision.HIGHEST)), 0.2)
        t = _lrelu(_ln_rows(jnp.dot(t, w22_ref[...], preferred_element_type=jnp.float32, precision=jax.lax.Precision.HIGHEST)), 0.2)
        t = _ln_rows(jnp.dot(t, w23_ref[...], preferred_element_type=jnp.float32, precision=jax.lax.Precision.HIGHEST))
        h3 = _lrelu(t + jnp.dot(h2, p2m_ref[...], preferred_element_type=jnp.float32, precision=jax.lax.Precision.HIGHEST), 0.2)

        out_ref[...] = (jnp.dot(h3, fct_ref[...], preferred_element_type=jnp.float32, precision=jax.lax.Precision.HIGHEST)
                        + fcb_ref[...])

    return pl.pallas_call(
        body,
        out_shape=jax.ShapeDtypeStruct((b, out_dim), jnp.float32),
    )(p1, p2, env2, wm1, wm2, gm, bm, d0w1a, d0w1b, d0w2t, d0w3t, p0m, p0e,
      d1w1t, d1w2t, d1w3t, d2w1t, d2w2t, d2w3t, p2map, fct, fcb)


def _prep_layer_weights(aw1, aw2, uw, rw, cin, rm):
    eff = cin - 3 if rm else cin
    w1 = jnp.transpose(aw1[:, :eff])
    w2 = jnp.transpose(aw1[:, eff:2 * eff])
    w3 = jnp.transpose(aw1[:, 2 * eff:])
    uwt = jnp.transpose(uw)
    rwt = jnp.transpose(rw) if rw is not None else None
    if rm:
        pad = jnp.zeros((3, 64), jnp.float32)
        w1 = jnp.concatenate([pad, w1], axis=0)
        w2 = jnp.concatenate([pad, w2], axis=0)
        padc = jnp.zeros((3, uwt.shape[1]), jnp.float32)
        uwt = jnp.concatenate([padc, uwt], axis=0)
        if rwt is not None:
            rwt = jnp.concatenate([jnp.zeros((3, rwt.shape[1]), jnp.float32),
                                   rwt], axis=0)
    return w1, w2, w3, aw2, uwt, rwt


def kernel(x, env, idx_base, mc0_aw1, mc0_aw2, mc0_uw, mc0_bg, mc0_bb, mc0_rw,
           mc1_aw1, mc1_aw2, mc1_uw, mc1_bg, mc1_bb,
           mc2_aw1, mc2_aw2, mc2_uw, mc2_bg, mc2_bb, mc2_rw,
           mc3_aw1, mc3_aw2, mc3_uw, mc3_bg, mc3_bb,
           conv_w, conv_bg, conv_bb, mrg_w, mrg_bg, mrg_bb,
           dec0_w1, dec0_w2, dec0_w3, dec1_w1, dec1_w2, dec1_w3,
           dec2_w1, dec2_w2, dec2_w3, fc_w, fc_b):
    b, cin0, n = x.shape
    emb = conv_w.shape[0]
    out_dim = fc_w.shape[0]
    tb = 2 if b % 2 == 0 else 1
    tb6 = 8 if b % 8 == 0 else 1
    bnk = b * n * _K

    xt = jnp.transpose(x, (0, 2, 1))

    w0 = _prep_layer_weights(mc0_aw1, mc0_aw2, mc0_uw, mc0_rw, cin0, True)
    w1 = _prep_layer_weights(mc1_aw1, mc1_aw2, mc1_uw, None, 32, False)
    w2 = _prep_layer_weights(mc2_aw1, mc2_aw2, mc2_uw, mc2_rw, 32, False)
    w3 = _prep_layer_weights(mc3_aw1, mc3_aw2, mc3_uw, None, 64, False)

    r2 = lambda v: v.reshape(1, -1)

    p0 = _layer_first_call(xt, w0, tb, n, 32)
    upd0, res0, s10, s20 = p0
    xc0, upd1, res1, s11, s21 = _layer_mid_call(
        (upd0, res0, s10, s20), r2(mc0_bg), r2(mc0_bb), w1, tb, n, 32,
        False, bnk)
    xc1, upd2, res2, s12, s22 = _layer_mid_call(
        (upd1, res1, s11, s21), r2(mc1_bg), r2(mc1_bb), w2, tb, n, 64,
        True, bnk)
    xc2, upd3, res3, s13, s23 = _layer_mid_call(
        (upd2, res2, s12, s22), r2(mc2_bg), r2(mc2_bb), w3, tb, n, 64,
        False, bnk)

    cw = jnp.transpose(conv_w)
    wcs = (cw[0:32], cw[32:64], cw[64:128], cw[128:192])
    y, ys1, ys2 = _conv_call((xc0, xc1, xc2), (upd3, res3, s13, s23),
                             r2(mc3_bg), r2(mc3_bb), wcs, tb, n, emb, bnk)

    p1, p2 = _pool_call(y, ys1, ys2, r2(conv_bg), r2(conv_bb), tb6, n, emb,
                        b * n)

    mrg_t = jnp.transpose(mrg_w)
    wm1, wm2 = mrg_t[:emb], mrg_t[emb:]
    d0w1t = jnp.transpose(dec0_w1)
    d0w1a, d0w1b = d0w1t[:emb], d0w1t[emb:]
    di0, do0 = dec0_w1.shape[1], dec0_w1.shape[0]
    p0full = (np.arange(di0)[:, None]
              == (np.arange(do0)[None, :] * di0) // do0).astype(np.float32)
    p0m = jnp.asarray(p0full[:emb])
    p0e = jnp.asarray(p0full[emb:])
    di2, do2 = dec2_w1.shape[1], dec2_w1.shape[0]
    p2map = jnp.asarray((np.arange(di2)[:, None]
                         == (np.arange(do2)[None, :] * di2) // do2)
                        .astype(np.float32))

    return _head_call(
        p1, p2, env.reshape(-1, 1), wm1, wm2, r2(mrg_bg), r2(mrg_bb),
        d0w1a, d0w1b, jnp.transpose(dec0_w2), jnp.transpose(dec0_w3),
        p0m, p0e,
        jnp.transpose(dec1_w1), jnp.transpose(dec1_w2), jnp.transpose(dec1_w3),
        jnp.transpose(dec2_w1), jnp.transpose(dec2_w2), jnp.transpose(dec2_w3),
        p2map,
        jnp.transpose(fc_w), fc_b.reshape(1, -1), out_dim)


# speedup vs baseline: 1.5277x; 1.5083x over previous
import numpy as np
import jax
import jax.numpy as jnp
from jax.experimental import pallas as pl

_K = 5
_NEG = -3.0e38


def _lrelu(x, a):
    return jnp.where(x >= 0, x, a * x)


def _centers_row(n_rows):
    idx = jax.lax.broadcasted_iota(jnp.int32, (n_rows, 16), 1)
    return idx.astype(jnp.float32) * (5.0 / 15.0)


def _gather_lanes(xtt, idxk, n):
    cin = xtt.shape[0]
    perm = jnp.broadcast_to(idxk, (cin, n))
    res = None
    for lo in range(0, n, 128):
        hi = min(lo + 128, n)
        tbl = xtt[:, lo:hi]
        loc = jnp.clip(perm - lo, 0, hi - lo - 1)
        g = jnp.take_along_axis(tbl, loc, axis=1)
        res = g if res is None else jnp.where(perm >= lo, g, res)
    return res


def _mol_core(xt, w1p, w2p, w3, aw2row, uwtp):
    n, _ = xt.shape
    xtt = jnp.transpose(xt)
    xx = jnp.sum(xt * xt, axis=1, keepdims=True)
    s = jax.lax.dot_general(xt, xt, (((1,), (1,)), ((), ())),
                            preferred_element_type=jnp.float32,
                            precision=jax.lax.Precision.HIGHEST)
    pair = 2.0 * s - xx - jnp.transpose(xx)
    row = jax.lax.broadcasted_iota(jnp.int32, (n, n), 0)
    centw1 = jnp.dot(xt, w1p, preferred_element_type=jnp.float32, precision=jax.lax.Precision.HIGHEST)
    cent_row = _centers_row(n)

    work = pair
    neighs, logits = [], []
    for _ in range(_K):
        m = jnp.max(work, axis=0, keepdims=True)
        is_max = work >= m
        idxk = jnp.min(jnp.where(is_max, row, jnp.int32(2 ** 30)),
                       axis=0, keepdims=True)
        sel = row == idxk
        work = jnp.where(sel, _NEG, work)
        neight = _gather_lanes(xtt, idxk, n)
        dist = jnp.sqrt(jnp.clip(-jnp.transpose(m), 1e-12, None))
        rbf = jnp.clip(jnp.exp(-10.0 * (dist - cent_row) ** 2), 1e-10, 1.0)
        nw2 = jax.lax.dot_general(neight, w2p, (((0,), (0,)), ((), ())),
                                  preferred_element_type=jnp.float32,
                                  precision=jax.lax.Precision.HIGHEST)
        h = _lrelu(centw1 + nw2
                   + jnp.dot(rbf, w3, preferred_element_type=jnp.float32, precision=jax.lax.Precision.HIGHEST), 0.2)
        logits.append(jnp.sum(h * aw2row, axis=1, keepdims=True))
        neighs.append(neight)

    mx = logits[0]
    for k in range(1, _K):
        mx = jnp.maximum(mx, logits[k])
    es = [jnp.exp(l - mx) for l in logits]
    z = es[0]
    for k in range(1, _K):
        z = z + es[k]

    upds = []
    for k in range(_K):
        att = es[k] / z
        nu = jax.lax.dot_general(neighs[k], uwtp, (((0,), (0,)), ((), ())),
                                 preferred_element_type=jnp.float32,
                                 precision=jax.lax.Precision.HIGHEST)
        upds.append(att * nu)
    mean_neigh_t = neighs[0]
    for k in range(1, _K):
        mean_neigh_t = mean_neigh_t + neighs[k]
    return upds, mean_neigh_t / float(_K)


def _bn_stats(s1, s2, count):
    mu = jnp.mean(s1, axis=0, keepdims=True) / count
    ex2 = jnp.mean(s2, axis=0, keepdims=True) / count
    var = ex2 - mu * mu
    rstd = 1.0 / jnp.sqrt(var + 1e-5)
    return mu, rstd


def _finalize_prev(upd_k, res_prev, mu, rstd, g, b):
    acc = None
    for k in range(_K):
        u = _lrelu((upd_k[k] - mu) * rstd * g + b, 0.02)
        acc = u if acc is None else acc + u
    return acc / float(_K) + 0.1 * res_prev


def _emit_layer(t, xt, wrefs, has_rw, upd_out, res_out, s1_out, s2_out, cout):
    w1p, w2p, w3, aw2, uwtp, rwtp = wrefs
    upds, mn = _mol_core(xt, w1p, w2p, w3, aw2, uwtp)
    ssum, ssq = None, None
    for k in range(_K):
        upd_out[t, k] = upds[k]
        cs = jnp.sum(upds[k], axis=0, keepdims=True)
        cq = jnp.sum(upds[k] * upds[k], axis=0, keepdims=True)
        ssum = cs if ssum is None else ssum + cs
        ssq = cq if ssq is None else ssq + cq
    if has_rw:
        res_out[t] = jax.lax.dot_general(
            mn, rwtp, (((0,), (0,)), ((), ())),
            preferred_element_type=jnp.float32,
            precision=jax.lax.Precision.HIGHEST)
    else:
        res_out[t] = jnp.transpose(mn)
    s1_out[...] += jnp.broadcast_to(ssum, (8, cout))
    s2_out[...] += jnp.broadcast_to(ssq, (8, cout))


def _layer_first_call(xt, w, tb, n, cout):
    b = xt.shape[0]
    w1p, w2p, w3, aw2, uwtp, rwtp = w
    cin = xt.shape[2]

    def body(x_ref, w1_ref, w2_ref, w3_ref, aw2_ref, uw_ref, rw_ref,
             upd_out, res_out, s1_out, s2_out):
        step = pl.program_id(0)

        @pl.when(step == 0)
        def _():
            s1_out[...] = jnp.zeros((8, cout), jnp.float32)
            s2_out[...] = jnp.zeros((8, cout), jnp.float32)

        wrefs = (w1_ref[...], w2_ref[...], w3_ref[...], aw2_ref[...],
                 uw_ref[...], rw_ref[...])
        for t in range(tb):
            _emit_layer(t, x_ref[t], wrefs, True, upd_out, res_out,
                        s1_out, s2_out, cout)

    grid = (b // tb,)
    const = lambda i: (0, 0)
    return pl.pallas_call(
        body,
        grid=grid,
        in_specs=[
            pl.BlockSpec((tb, n, cin), lambda i: (i, 0, 0)),
            pl.BlockSpec((cin, 64), const),
            pl.BlockSpec((cin, 64), const),
            pl.BlockSpec((16, 64), const),
            pl.BlockSpec((1, 64), const),
            pl.BlockSpec((cin, cout), const),
            pl.BlockSpec((cin, cout), const),
        ],
        out_specs=[
            pl.BlockSpec((tb, _K, n, cout), lambda i: (i, 0, 0, 0)),
            pl.BlockSpec((tb, n, cout), lambda i: (i, 0, 0)),
            pl.BlockSpec((8, cout), const),
            pl.BlockSpec((8, cout), const),
        ],
        out_shape=[
            jax.ShapeDtypeStruct((b, _K, n, cout), jnp.float32),
            jax.ShapeDtypeStruct((b, n, cout), jnp.float32),
            jax.ShapeDtypeStruct((8, cout), jnp.float32),
            jax.ShapeDtypeStruct((8, cout), jnp.float32),
        ],
    )(xt, *w)


def _layer_mid_call(prev, gp, bp, w, tb, n, cout, has_rw, bnk):
    upd_p, res_p, s1_p, s2_p = prev
    b = upd_p.shape[0]
    cp = upd_p.shape[3]
    w1p, w2p, w3, aw2, uwtp, rwtp = w
    cin = cp

    def body(updp_ref, resp_ref, s1p_ref, s2p_ref, gp_ref, bp_ref,
             w1_ref, w2_ref, w3_ref, aw2_ref, uw_ref, rw_ref,
             xc_out, upd_out, res_out, s1_out, s2_out):
        step = pl.program_id(0)

        @pl.when(step == 0)
        def _():
            s1_out[...] = jnp.zeros((8, cout), jnp.float32)
            s2_out[...] = jnp.zeros((8, cout), jnp.float32)

        mu, rstd = _bn_stats(s1p_ref[...], s2p_ref[...], float(bnk))
        gpv, bpv = gp_ref[...], bp_ref[...]
        wrefs = (w1_ref[...], w2_ref[...], w3_ref[...], aw2_ref[...],
                 uw_ref[...], rw_ref[...] if rw_ref is not None else None)
        for t in range(tb):
            upd_k = [updp_ref[t, k] for k in range(_K)]
            cur = _finalize_prev(upd_k, resp_ref[t], mu, rstd, gpv, bpv)
            xc_out[t] = cur
            _emit_layer(t, cur, wrefs, has_rw, upd_out, res_out,
                        s1_out, s2_out, cout)

    grid = (b // tb,)
    const = lambda i: (0, 0)
    in_specs = [
        pl.BlockSpec((tb, _K, n, cp), lambda i: (i, 0, 0, 0)),
        pl.BlockSpec((tb, n, cp), lambda i: (i, 0, 0)),
        pl.BlockSpec((8, cp), const),
        pl.BlockSpec((8, cp), const),
        pl.BlockSpec((1, cp), const),
        pl.BlockSpec((1, cp), const),
        pl.BlockSpec((cin, 64), const),
        pl.BlockSpec((cin, 64), const),
        pl.BlockSpec((16, 64), const),
        pl.BlockSpec((1, 64), const),
        pl.BlockSpec((cin, cout), const),
    ]
    args = [upd_p, res_p, s1_p, s2_p, gp, bp, w1p, w2p, w3, aw2, uwtp]
    if has_rw:
        in_specs.append(pl.BlockSpec((cin, cout), const))
        args.append(rwtp)
        fn = body
    else:
        def fn(updp_ref, resp_ref, s1p_ref, s2p_ref, gp_ref, bp_ref,
               w1_ref, w2_ref, w3_ref, aw2_ref, uw_ref,
               xc_out, upd_out, res_out, s1_out, s2_out):
            body(updp_ref, resp_ref, s1p_ref, s2p_ref, gp_ref, bp_ref,
                 w1_ref, w2_ref, w3_ref, aw2_ref, uw_ref, None,
                 xc_out, upd_out, res_out, s1_out, s2_out)

    return pl.pallas_call(
        fn,
        grid=grid,
        in_specs=in_specs,
        out_specs=[
            pl.BlockSpec((tb, n, cp), lambda i: (i, 0, 0)),
            pl.BlockSpec((tb, _K, n, cout), lambda i: (i, 0, 0, 0)),
            pl.BlockSpec((tb, n, cout), lambda i: (i, 0, 0)),
            pl.BlockSpec((8, cout), const),
            pl.BlockSpec((8, cout), const),
        ],
        out_shape=[
            jax.ShapeDtypeStruct((b, n, cp), jnp.float32),
            jax.ShapeDtypeStruct((b, _K, n, cout), jnp.float32),
            jax.ShapeDtypeStruct((b, n, cout), jnp.float32),
            jax.ShapeDtypeStruct((8, cout), jnp.float32),
            jax.ShapeDtypeStruct((8, cout), jnp.float32),
        ],
    )(*args)


def _conv_call(xcs, prev, gp, bp, wcs, tb, n, emb, bnk):
    xc0, xc1, xc2 = xcs
    upd_p, res_p, s1_p, s2_p = prev
    b = upd_p.shape[0]
    cp = upd_p.shape[3]
    c0, c1, c2 = xc0.shape[2], xc1.shape[2], xc2.shape[2]
    wc0, wc1, wc2, wc3 = wcs

    def body(xc0_ref, xc1_ref, xc2_ref, updp_ref, resp_ref, s1p_ref, s2p_ref,
             gp_ref, bp_ref, wc0_ref, wc1_ref, wc2_ref, wc3_ref,
             y_out, ys1_out, ys2_out):
        step = pl.program_id(0)

        @pl.when(step == 0)
        def _():
            ys1_out[...] = jnp.zeros((8, emb), jnp.float32)
            ys2_out[...] = jnp.zeros((8, emb), jnp.float32)

        mu, rstd = _bn_stats(s1p_ref[...], s2p_ref[...], float(bnk))
        gpv, bpv = gp_ref[...], bp_ref[...]
        for t in range(tb):
            upd_k = [updp_ref[t, k] for k in range(_K)]
            cur3 = _finalize_prev(upd_k, resp_ref[t], mu, rstd, gpv, bpv)
            y = (jnp.dot(xc0_ref[t], wc0_ref[...], preferred_element_type=jnp.float32, precision=jax.lax.Precision.HIGHEST)
                 + jnp.dot(xc1_ref[t], wc1_ref[...], preferred_element_type=jnp.float32, precision=jax.lax.Precision.HIGHEST)
                 + jnp.dot(xc2_ref[t], wc2_ref[...], preferred_element_type=jnp.float32, precision=jax.lax.Precision.HIGHEST)
                 + jnp.dot(cur3, wc3_ref[...], preferred_element_type=jnp.float32, precision=jax.lax.Precision.HIGHEST))
            y_out[t] = y
            ys1_out[...] += jnp.broadcast_to(
                jnp.sum(y, axis=0, keepdims=True), (8, emb))
            ys2_out[...] += jnp.broadcast_to(
                jnp.sum(y * y, axis=0, keepdims=True), (8, emb))

    grid = (b // tb,)
    const = lambda i: (0, 0)
    return pl.pallas_call(
        body,
        grid=grid,
        in_specs=[
            pl.BlockSpec((tb, n, c0), lambda i: (i, 0, 0)),
            pl.BlockSpec((tb, n, c1), lambda i: (i, 0, 0)),
            pl.BlockSpec((tb, n, c2), lambda i: (i, 0, 0)),
            pl.BlockSpec((tb, _K, n, cp), lambda i: (i, 0, 0, 0)),
            pl.BlockSpec((tb, n, cp), lambda i: (i, 0, 0)),
            pl.BlockSpec((8, cp), const),
            pl.BlockSpec((8, cp), const),
            pl.BlockSpec((1, cp), const),
            pl.BlockSpec((1, cp), const),
            pl.BlockSpec((c0, emb), const),
            pl.BlockSpec((c1, emb), const),
            pl.BlockSpec((c2, emb), const),
            pl.BlockSpec((cp, emb), const),
        ],
        out_specs=[
            pl.BlockSpec((tb, n, emb), lambda i: (i, 0, 0)),
            pl.BlockSpec((8, emb), const),
            pl.BlockSpec((8, emb), const),
        ],
        out_shape=[
            jax.ShapeDtypeStruct((b, n, emb), jnp.float32),
            jax.ShapeDtypeStruct((8, emb), jnp.float32),
            jax.ShapeDtypeStruct((8, emb), jnp.float32),
        ],
    )(xc0, xc1, xc2, upd_p, res_p, s1_p, s2_p, gp, bp, wc0, wc1, wc2, wc3)


def _pool_call(y, ys1, ys2, g, bb, tb, n, emb, bn_count):
    b = y.shape[0]

    def body(y_ref, s1_ref, s2_ref, g_ref, b_ref, p1_out, p2_out):
        mu, rstd = _bn_stats(s1_ref[...], s2_ref[...], float(bn_count))
        gv, bv = g_ref[...], b_ref[...]
        for t in range(tb):
            z = _lrelu((y_ref[t] - mu) * rstd * gv + bv, 0.2)
            p1_out[pl.ds(t, 1), :] = jnp.max(z, axis=0, keepdims=True)
            p2_out[pl.ds(t, 1), :] = jnp.mean(z, axis=0, keepdims=True)

    grid = (b // tb,)
    const = lambda i: (0, 0)
    return pl.pallas_call(
        body,
        grid=grid,
        in_specs=[
            pl.BlockSpec((tb, n, emb), lambda i: (i, 0, 0)),
            pl.BlockSpec((8, emb), const),
            pl.BlockSpec((8, emb), const),
            pl.BlockSpec((1, emb), const),
            pl.BlockSpec((1, emb), const),
        ],
        out_specs=[
            pl.BlockSpec((tb, emb), lambda i: (i, 0)),
            pl.BlockSpec((tb, emb), lambda i: (i, 0)),
        ],
        out_shape=[
            jax.ShapeDtypeStruct((b, emb), jnp.float32),
            jax.ShapeDtypeStruct((b, emb), jnp.float32),
        ],
    )(y, ys1, ys2, g, bb)


def _ln_rows(v):
    mu = jnp.mean(v, axis=1, keepdims=True)
    var = jnp.mean((v - mu) ** 2, axis=1, keepdims=True)
    return (v - mu) / jnp.sqrt(var + 1e-5)


def _head_call(p1, p2, env2, wm1, wm2, gm, bm,
               d0w1a, d0w1b, d0w2t, d0w3t, p0m, p0e,
               d1w1t, d1w2t, d1w3t, d2w1t, d2w2t, d2w3t, p2map,
               fct, fcb, out_dim):
    b = p1.shape[0]

    def body(p1_ref, p2_ref, e_ref, wm1_ref, wm2_ref, gm_ref, bm_ref,
             a_ref, b1_ref, w02_ref, w03_ref, p0m_ref, p0e_ref,
             w11_ref, w12_ref, w13_ref, w21_ref, w22_ref, w23_ref, p2m_ref,
             fct_ref, fcb_ref, out_ref):
        m0 = (jnp.dot(p1_ref[...], wm1_ref[...], preferred_element_type=jnp.float32, precision=jax.lax.Precision.HIGHEST)
              + jnp.dot(p2_ref[...], wm2_ref[...], preferred_element_type=jnp.float32, precision=jax.lax.Precision.HIGHEST))
        mu = jnp.mean(m0, axis=0, keepdims=True)
        var = jnp.mean((m0 - mu) ** 2, axis=0, keepdims=True)
        m = _lrelu((m0 - mu) / jnp.sqrt(var + 1e-5) * gm_ref[...] + bm_ref[...], 0.2)
        e = e_ref[...]

        t = _lrelu(_ln_rows(jnp.dot(m, a_ref[...], preferred_element_type=jnp.float32, precision=jax.lax.Precision.HIGHEST)
                            + e * b1_ref[...]), 0.2)
        t = _lrelu(_ln_rows(jnp.dot(t, w02_ref[...], preferred_element_type=jnp.float32, precision=jax.lax.Precision.HIGHEST)), 0.2)
        t = _ln_rows(jnp.dot(t, w03_ref[...], preferred_element_type=jnp.float32, precision=jax.lax.Precision.HIGHEST))
        idn = jnp.dot(m, p0m_ref[...], preferred_element_type=jnp.float32, precision=jax.lax.Precision.HIGHEST) + e * p0e_ref[...]
        h1 = _lrelu(t + idn, 0.2)

        t = _lrelu(_ln_rows(jnp.dot(h1, w11_ref[...], preferred_element_type=jnp.float32, precision=jax.lax.Precision.HIGHEST)), 0.2)
        t = _lrelu(_ln_rows(jnp.dot(t, w12_ref[...], preferred_element_type=jnp.float32, precision=jax.lax.Precision.HIGHEST)), 0.2)
        t = _ln_rows(jnp.dot(t, w13_ref[...], preferred_element_type=jnp.float32, precision=jax.lax.Precision.HIGHEST))
        h2 = _lrelu(t + h1, 0.2)

        t = _lrelu(_ln_rows(jnp.dot(h2, w21_ref[...], preferred_element_type=jnp.float32, precision=jax.lax.Precision.HIGHEST)), 0.2)
        t = _lrelu(_ln_rows(jnp.dot(t, w22_ref[...], preferred_element_type=jnp.float32, precision=jax.lax.Precision.HIGHEST)), 0.2)
        t = _ln_rows(jnp.dot(t, w23_ref[...], preferred_element_type=jnp.float32, precision=jax.lax.Precision.HIGHEST))
        h3 = _lrelu(t + jnp.dot(h2, p2m_ref[...], preferred_element_type=jnp.float32, precision=jax.lax.Precision.HIGHEST), 0.2)

        out_ref[...] = (jnp.dot(h3, fct_ref[...], preferred_element_type=jnp.float32, precision=jax.lax.Precision.HIGHEST)
                        + fcb_ref[...])

    return pl.pallas_call(
        body,
        out_shape=jax.ShapeDtypeStruct((b, out_dim), jnp.float32),
    )(p1, p2, env2, wm1, wm2, gm, bm, d0w1a, d0w1b, d0w2t, d0w3t, p0m, p0e,
      d1w1t, d1w2t, d1w3t, d2w1t, d2w2t, d2w3t, p2map, fct, fcb)


def _prep_layer_weights(aw1, aw2, uw, rw, cin, rm):
    eff = cin - 3 if rm else cin
    w1 = jnp.transpose(aw1[:, :eff])
    w2 = jnp.transpose(aw1[:, eff:2 * eff])
    w3 = jnp.transpose(aw1[:, 2 * eff:])
    uwt = jnp.transpose(uw)
    rwt = jnp.transpose(rw) if rw is not None else None
    if rm:
        pad = jnp.zeros((3, 64), jnp.float32)
        w1 = jnp.concatenate([pad, w1], axis=0)
        w2 = jnp.concatenate([pad, w2], axis=0)
        padc = jnp.zeros((3, uwt.shape[1]), jnp.float32)
        uwt = jnp.concatenate([padc, uwt], axis=0)
        if rwt is not None:
            rwt = jnp.concatenate([jnp.zeros((3, rwt.shape[1]), jnp.float32),
                                   rwt], axis=0)
    return w1, w2, w3, aw2, uwt, rwt


def kernel(x, env, idx_base, mc0_aw1, mc0_aw2, mc0_uw, mc0_bg, mc0_bb, mc0_rw,
           mc1_aw1, mc1_aw2, mc1_uw, mc1_bg, mc1_bb,
           mc2_aw1, mc2_aw2, mc2_uw, mc2_bg, mc2_bb, mc2_rw,
           mc3_aw1, mc3_aw2, mc3_uw, mc3_bg, mc3_bb,
           conv_w, conv_bg, conv_bb, mrg_w, mrg_bg, mrg_bb,
           dec0_w1, dec0_w2, dec0_w3, dec1_w1, dec1_w2, dec1_w3,
           dec2_w1, dec2_w2, dec2_w3, fc_w, fc_b):
    b, cin0, n = x.shape
    emb = conv_w.shape[0]
    out_dim = fc_w.shape[0]
    tb = 2 if b % 2 == 0 else 1
    tb6 = 8 if b % 8 == 0 else 1
    bnk = b * n * _K

    xt = jnp.transpose(x, (0, 2, 1))

    w0 = _prep_layer_weights(mc0_aw1, mc0_aw2, mc0_uw, mc0_rw, cin0, True)
    w1 = _prep_layer_weights(mc1_aw1, mc1_aw2, mc1_uw, None, 32, False)
    w2 = _prep_layer_weights(mc2_aw1, mc2_aw2, mc2_uw, mc2_rw, 32, False)
    w3 = _prep_layer_weights(mc3_aw1, mc3_aw2, mc3_uw, None, 64, False)

    r2 = lambda v: v.reshape(1, -1)

    p0 = _layer_first_call(xt, w0, tb, n, 32)
    upd0, res0, s10, s20 = p0
    xc0, upd1, res1, s11, s21 = _layer_mid_call(
        (upd0, res0, s10, s20), r2(mc0_bg), r2(mc0_bb), w1, tb, n, 32,
        False, bnk)
    xc1, upd2, res2, s12, s22 = _layer_mid_call(
        (upd1, res1, s11, s21), r2(mc1_bg), r2(mc1_bb), w2, tb, n, 64,
        True, bnk)
    xc2, upd3, res3, s13, s23 = _layer_mid_call(
        (upd2, res2, s12, s22), r2(mc2_bg), r2(mc2_bb), w3, tb, n, 64,
        False, bnk)

    cw = jnp.transpose(conv_w)
    wcs = (cw[0:32], cw[32:64], cw[64:128], cw[128:192])
    y, ys1, ys2 = _conv_call((xc0, xc1, xc2), (upd3, res3, s13, s23),
                             r2(mc3_bg), r2(mc3_bb), wcs, tb, n, emb, bnk)

    p1, p2 = _pool_call(y, ys1, ys2, r2(conv_bg), r2(conv_bb), tb6, n, emb,
                        b * n)

    mrg_t = jnp.transpose(mrg_w)
    wm1, wm2 = mrg_t[:emb], mrg_t[emb:]
    d0w1t = jnp.transpose(dec0_w1)
    d0w1a, d0w1b = d0w1t[:emb], d0w1t[emb:]
    di0, do0 = dec0_w1.shape[1], dec0_w1.shape[0]
    p0full = (np.arange(di0)[:, None]
              == (np.arange(do0)[None, :] * di0) // do0).astype(np.float32)
    p0m = jnp.asarray(p0full[:emb])
    p0e = jnp.asarray(p0full[emb:])
    di2, do2 = dec2_w1.shape[1], dec2_w1.shape[0]
    p2map = jnp.asarray((np.arange(di2)[:, None]
                         == (np.arange(do2)[None, :] * di2) // do2)
                        .astype(np.float32))

    return _head_call(
        p1, p2, env.reshape(-1, 1), wm1, wm2, r2(mrg_bg), r2(mrg_bb),
        d0w1a, d0w1b, jnp.transpose(dec0_w2), jnp.transpose(dec0_w3),
        p0m, p0e,
        jnp.transpose(dec1_w1), jnp.transpose(dec1_w2), jnp.transpose(dec1_w3),
        jnp.transpose(dec2_w1), jnp.transpose(dec2_w2), jnp.transpose(dec2_w3),
        p2map,
        jnp.transpose(fc_w), fc_b.reshape(1, -1), out_dim)


# speedup vs baseline: 2.0335x; 1.3311x over previous
import numpy as np
import jax
import jax.numpy as jnp
from jax.experimental import pallas as pl

_K = 5
_NEG = -3.0e38


def _lrelu(x, a):
    return jnp.where(x >= 0, x, a * x)


def _centers_row(n_rows):
    idx = jax.lax.broadcasted_iota(jnp.int32, (n_rows, 16), 1)
    return idx.astype(jnp.float32) * (5.0 / 15.0)


def _gather_lanes(xtt, idxk, n):
    cin = xtt.shape[0]
    perm = jnp.broadcast_to(idxk, (cin, n))
    res = None
    for lo in range(0, n, 128):
        hi = min(lo + 128, n)
        tbl = xtt[:, lo:hi]
        loc = jnp.clip(perm - lo, 0, hi - lo - 1)
        g = jnp.take_along_axis(tbl, loc, axis=1)
        res = g if res is None else jnp.where(perm >= lo, g, res)
    return res


def _mol_core(xt, w1p, w23u, aw2row, cout):
    n, _ = xt.shape
    xtt = jnp.transpose(xt)
    xx = jnp.sum(xt * xt, axis=1, keepdims=True)
    s = jax.lax.dot_general(xt, xt, (((1,), (1,)), ((), ())),
                            preferred_element_type=jnp.float32,
                            precision=jax.lax.Precision.HIGHEST)
    pair = 2.0 * s - xx - jnp.transpose(xx)
    row = jax.lax.broadcasted_iota(jnp.int32, (n, n), 0)
    centw1 = jnp.dot(xt, w1p, preferred_element_type=jnp.float32, precision=jax.lax.Precision.HIGHEST)

    ci = jax.lax.broadcasted_iota(jnp.int32, (16, n), 0) \
        .astype(jnp.float32) * (5.0 / 15.0)

    work = pair
    neighs, logits, nus = [], [], []
    for _ in range(_K):
        m = jnp.max(work, axis=0, keepdims=True)
        is_max = work >= m
        idxk = jnp.min(jnp.where(is_max, row, jnp.int32(2 ** 30)),
                       axis=0, keepdims=True)
        sel = row == idxk
        work = jnp.where(sel, _NEG, work)
        neight = _gather_lanes(xtt, idxk, n)
        dist = jnp.sqrt(jnp.clip(-m, 1e-12, None))
        rbft = jnp.clip(jnp.exp(-10.0 * (dist - ci) ** 2), 1e-10, 1.0)
        att_t = jnp.concatenate([neight, rbft], axis=0)
        hn = jax.lax.dot_general(att_t, w23u, (((0,), (0,)), ((), ())),
                                 preferred_element_type=jnp.float32,
                                 precision=jax.lax.Precision.HIGHEST)
        h = _lrelu(centw1 + hn[:, :64], 0.2)
        logits.append(jnp.sum(h * aw2row, axis=1, keepdims=True))
        nus.append(hn[:, 64:64 + cout])
        neighs.append(neight)

    mx = logits[0]
    for k in range(1, _K):
        mx = jnp.maximum(mx, logits[k])
    es = [jnp.exp(l - mx) for l in logits]
    z = es[0]
    for k in range(1, _K):
        z = z + es[k]

    upds = []
    for k in range(_K):
        att = es[k] / z
        upds.append(att * nus[k])
    mean_neigh_t = neighs[0]
    for k in range(1, _K):
        mean_neigh_t = mean_neigh_t + neighs[k]
    return upds, mean_neigh_t / float(_K)


def _bn_stats(s1, s2, count):
    mu = jnp.mean(s1, axis=0, keepdims=True) / count
    ex2 = jnp.mean(s2, axis=0, keepdims=True) / count
    var = ex2 - mu * mu
    rstd = 1.0 / jnp.sqrt(var + 1e-5)
    return mu, rstd


def _finalize_prev(upd_k, res_prev, mu, rstd, g, b):
    acc = None
    for k in range(_K):
        u = _lrelu((upd_k[k] - mu) * rstd * g + b, 0.02)
        acc = u if acc is None else acc + u
    return acc / float(_K) + 0.1 * res_prev


def _emit_layer(t, xt, wrefs, has_rw, upd_out, res_out, s1_out, s2_out, cout):
    w1p, w23u, aw2, rwtp = wrefs
    upds, mn = _mol_core(xt, w1p, w23u, aw2, cout)
    ssum, ssq = None, None
    for k in range(_K):
        upd_out[t, k] = upds[k]
        cs = jnp.sum(upds[k], axis=0, keepdims=True)
        cq = jnp.sum(upds[k] * upds[k], axis=0, keepdims=True)
        ssum = cs if ssum is None else ssum + cs
        ssq = cq if ssq is None else ssq + cq
    if has_rw:
        res_out[t] = jax.lax.dot_general(
            mn, rwtp, (((0,), (0,)), ((), ())),
            preferred_element_type=jnp.float32,
            precision=jax.lax.Precision.HIGHEST)
    else:
        res_out[t] = jnp.transpose(mn)
    s1_out[...] += jnp.broadcast_to(ssum, (8, cout))
    s2_out[...] += jnp.broadcast_to(ssq, (8, cout))


def _layer_first_call(xt, w, tb, n, cout):
    b = xt.shape[0]
    w1p, w23u, aw2, rwtp = w
    cin = xt.shape[2]

    def body(x_ref, w1_ref, w23_ref, aw2_ref, rw_ref,
             upd_out, res_out, s1_out, s2_out):
        step = pl.program_id(0)

        @pl.when(step == 0)
        def _():
            s1_out[...] = jnp.zeros((8, cout), jnp.float32)
            s2_out[...] = jnp.zeros((8, cout), jnp.float32)

        wrefs = (w1_ref[...], w23_ref[...], aw2_ref[...], rw_ref[...])
        for t in range(tb):
            _emit_layer(t, x_ref[t], wrefs, True, upd_out, res_out,
                        s1_out, s2_out, cout)

    grid = (b // tb,)
    const = lambda i: (0, 0)
    return pl.pallas_call(
        body,
        grid=grid,
        in_specs=[
            pl.BlockSpec((tb, n, cin), lambda i: (i, 0, 0)),
            pl.BlockSpec((cin, 64), const),
            pl.BlockSpec((cin + 16, 64 + cout), const),
            pl.BlockSpec((1, 64), const),
            pl.BlockSpec((cin, cout), const),
        ],
        out_specs=[
            pl.BlockSpec((tb, _K, n, cout), lambda i: (i, 0, 0, 0)),
            pl.BlockSpec((tb, n, cout), lambda i: (i, 0, 0)),
            pl.BlockSpec((8, cout), const),
            pl.BlockSpec((8, cout), const),
        ],
        out_shape=[
            jax.ShapeDtypeStruct((b, _K, n, cout), jnp.float32),
            jax.ShapeDtypeStruct((b, n, cout), jnp.float32),
            jax.ShapeDtypeStruct((8, cout), jnp.float32),
            jax.ShapeDtypeStruct((8, cout), jnp.float32),
        ],
    )(xt, *w)


def _layer_mid_call(prev, gp, bp, w, tb, n, cout, has_rw, bnk):
    upd_p, res_p, s1_p, s2_p = prev
    b = upd_p.shape[0]
    cp = upd_p.shape[3]
    w1p, w23u, aw2, rwtp = w
    cin = cp

    def body(updp_ref, resp_ref, s1p_ref, s2p_ref, gp_ref, bp_ref,
             w1_ref, w23_ref, aw2_ref, rw_ref,
             xc_out, upd_out, res_out, s1_out, s2_out):
        step = pl.program_id(0)

        @pl.when(step == 0)
        def _():
            s1_out[...] = jnp.zeros((8, cout), jnp.float32)
            s2_out[...] = jnp.zeros((8, cout), jnp.float32)

        mu, rstd = _bn_stats(s1p_ref[...], s2p_ref[...], float(bnk))
        gpv, bpv = gp_ref[...], bp_ref[...]
        wrefs = (w1_ref[...], w23_ref[...], aw2_ref[...],
                 rw_ref[...] if rw_ref is not None else None)
        for t in range(tb):
            upd_k = [updp_ref[t, k] for k in range(_K)]
            cur = _finalize_prev(upd_k, resp_ref[t], mu, rstd, gpv, bpv)
            xc_out[t] = cur
            _emit_layer(t, cur, wrefs, has_rw, upd_out, res_out,
                        s1_out, s2_out, cout)

    grid = (b // tb,)
    const = lambda i: (0, 0)
    in_specs = [
        pl.BlockSpec((tb, _K, n, cp), lambda i: (i, 0, 0, 0)),
        pl.BlockSpec((tb, n, cp), lambda i: (i, 0, 0)),
        pl.BlockSpec((8, cp), const),
        pl.BlockSpec((8, cp), const),
        pl.BlockSpec((1, cp), const),
        pl.BlockSpec((1, cp), const),
        pl.BlockSpec((cin, 64), const),
        pl.BlockSpec((cin + 16, 64 + cout), const),
        pl.BlockSpec((1, 64), const),
    ]
    args = [upd_p, res_p, s1_p, s2_p, gp, bp, w1p, w23u, aw2]
    if has_rw:
        in_specs.append(pl.BlockSpec((cin, cout), const))
        args.append(rwtp)
        fn = body
    else:
        def fn(updp_ref, resp_ref, s1p_ref, s2p_ref, gp_ref, bp_ref,
               w1_ref, w23_ref, aw2_ref,
               xc_out, upd_out, res_out, s1_out, s2_out):
            body(updp_ref, resp_ref, s1p_ref, s2p_ref, gp_ref, bp_ref,
                 w1_ref, w23_ref, aw2_ref, None,
                 xc_out, upd_out, res_out, s1_out, s2_out)

    return pl.pallas_call(
        fn,
        grid=grid,
        in_specs=in_specs,
        out_specs=[
            pl.BlockSpec((tb, n, cp), lambda i: (i, 0, 0)),
            pl.BlockSpec((tb, _K, n, cout), lambda i: (i, 0, 0, 0)),
            pl.BlockSpec((tb, n, cout), lambda i: (i, 0, 0)),
            pl.BlockSpec((8, cout), const),
            pl.BlockSpec((8, cout), const),
        ],
        out_shape=[
            jax.ShapeDtypeStruct((b, n, cp), jnp.float32),
            jax.ShapeDtypeStruct((b, _K, n, cout), jnp.float32),
            jax.ShapeDtypeStruct((b, n, cout), jnp.float32),
            jax.ShapeDtypeStruct((8, cout), jnp.float32),
            jax.ShapeDtypeStruct((8, cout), jnp.float32),
        ],
    )(*args)


def _conv_call(xcs, prev, gp, bp, wcs, tb, n, emb, bnk):
    xc0, xc1, xc2 = xcs
    upd_p, res_p, s1_p, s2_p = prev
    b = upd_p.shape[0]
    cp = upd_p.shape[3]
    c0, c1, c2 = xc0.shape[2], xc1.shape[2], xc2.shape[2]
    wc0, wc1, wc2, wc3 = wcs

    def body(xc0_ref, xc1_ref, xc2_ref, updp_ref, resp_ref, s1p_ref, s2p_ref,
             gp_ref, bp_ref, wc0_ref, wc1_ref, wc2_ref, wc3_ref,
             y_out, ys1_out, ys2_out):
        step = pl.program_id(0)

        @pl.when(step == 0)
        def _():
            ys1_out[...] = jnp.zeros((8, emb), jnp.float32)
            ys2_out[...] = jnp.zeros((8, emb), jnp.float32)

        mu, rstd = _bn_stats(s1p_ref[...], s2p_ref[...], float(bnk))
        gpv, bpv = gp_ref[...], bp_ref[...]
        for t in range(tb):
            upd_k = [updp_ref[t, k] for k in range(_K)]
            cur3 = _finalize_prev(upd_k, resp_ref[t], mu, rstd, gpv, bpv)
            y = (jnp.dot(xc0_ref[t], wc0_ref[...], preferred_element_type=jnp.float32, precision=jax.lax.Precision.HIGHEST)
                 + jnp.dot(xc1_ref[t], wc1_ref[...], preferred_element_type=jnp.float32, precision=jax.lax.Precision.HIGHEST)
                 + jnp.dot(xc2_ref[t], wc2_ref[...], preferred_element_type=jnp.float32, precision=jax.lax.Precision.HIGHEST)
                 + jnp.dot(cur3, wc3_ref[...], preferred_element_type=jnp.float32, precision=jax.lax.Precision.HIGHEST))
            y_out[t] = y
            ys1_out[...] += jnp.broadcast_to(
                jnp.sum(y, axis=0, keepdims=True), (8, emb))
            ys2_out[...] += jnp.broadcast_to(
                jnp.sum(y * y, axis=0, keepdims=True), (8, emb))

    grid = (b // tb,)
    const = lambda i: (0, 0)
    return pl.pallas_call(
        body,
        grid=grid,
        in_specs=[
            pl.BlockSpec((tb, n, c0), lambda i: (i, 0, 0)),
            pl.BlockSpec((tb, n, c1), lambda i: (i, 0, 0)),
            pl.BlockSpec((tb, n, c2), lambda i: (i, 0, 0)),
            pl.BlockSpec((tb, _K, n, cp), lambda i: (i, 0, 0, 0)),
            pl.BlockSpec((tb, n, cp), lambda i: (i, 0, 0)),
            pl.BlockSpec((8, cp), const),
            pl.BlockSpec((8, cp), const),
            pl.BlockSpec((1, cp), const),
            pl.BlockSpec((1, cp), const),
            pl.BlockSpec((c0, emb), const),
            pl.BlockSpec((c1, emb), const),
            pl.BlockSpec((c2, emb), const),
            pl.BlockSpec((cp, emb), const),
        ],
        out_specs=[
            pl.BlockSpec((tb, n, emb), lambda i: (i, 0, 0)),
            pl.BlockSpec((8, emb), const),
            pl.BlockSpec((8, emb), const),
        ],
        out_shape=[
            jax.ShapeDtypeStruct((b, n, emb), jnp.float32),
            jax.ShapeDtypeStruct((8, emb), jnp.float32),
            jax.ShapeDtypeStruct((8, emb), jnp.float32),
        ],
    )(xc0, xc1, xc2, upd_p, res_p, s1_p, s2_p, gp, bp, wc0, wc1, wc2, wc3)


def _pool_call(y, ys1, ys2, g, bb, tb, n, emb, bn_count):
    b = y.shape[0]

    def body(y_ref, s1_ref, s2_ref, g_ref, b_ref, p1_out, p2_out):
        mu, rstd = _bn_stats(s1_ref[...], s2_ref[...], float(bn_count))
        gv, bv = g_ref[...], b_ref[...]
        for t in range(tb):
            z = _lrelu((y_ref[t] - mu) * rstd * gv + bv, 0.2)
            p1_out[pl.ds(t, 1), :] = jnp.max(z, axis=0, keepdims=True)
            p2_out[pl.ds(t, 1), :] = jnp.mean(z, axis=0, keepdims=True)

    grid = (b // tb,)
    const = lambda i: (0, 0)
    return pl.pallas_call(
        body,
        grid=grid,
        in_specs=[
            pl.BlockSpec((tb, n, emb), lambda i: (i, 0, 0)),
            pl.BlockSpec((8, emb), const),
            pl.BlockSpec((8, emb), const),
            pl.BlockSpec((1, emb), const),
            pl.BlockSpec((1, emb), const),
        ],
        out_specs=[
            pl.BlockSpec((tb, emb), lambda i: (i, 0)),
            pl.BlockSpec((tb, emb), lambda i: (i, 0)),
        ],
        out_shape=[
            jax.ShapeDtypeStruct((b, emb), jnp.float32),
            jax.ShapeDtypeStruct((b, emb), jnp.float32),
        ],
    )(y, ys1, ys2, g, bb)


def _ln_rows(v):
    mu = jnp.mean(v, axis=1, keepdims=True)
    var = jnp.mean((v - mu) ** 2, axis=1, keepdims=True)
    return (v - mu) / jnp.sqrt(var + 1e-5)


def _head_call(p1, p2, env2, wm1, wm2, gm, bm,
               d0w1a, d0w1b, d0w2t, d0w3t, p0m, p0e,
               d1w1t, d1w2t, d1w3t, d2w1t, d2w2t, d2w3t, p2map,
               fct, fcb, out_dim):
    b = p1.shape[0]

    def body(p1_ref, p2_ref, e_ref, wm1_ref, wm2_ref, gm_ref, bm_ref,
             a_ref, b1_ref, w02_ref, w03_ref, p0m_ref, p0e_ref,
             w11_ref, w12_ref, w13_ref, w21_ref, w22_ref, w23_ref, p2m_ref,
             fct_ref, fcb_ref, out_ref):
        m0 = (jnp.dot(p1_ref[...], wm1_ref[...], preferred_element_type=jnp.float32, precision=jax.lax.Precision.HIGHEST)
              + jnp.dot(p2_ref[...], wm2_ref[...], preferred_element_type=jnp.float32, precision=jax.lax.Precision.HIGHEST))
        mu = jnp.mean(m0, axis=0, keepdims=True)
        var = jnp.mean((m0 - mu) ** 2, axis=0, keepdims=True)
        m = _lrelu((m0 - mu) / jnp.sqrt(var + 1e-5) * gm_ref[...] + bm_ref[...], 0.2)
        e = e_ref[...]

        t = _lrelu(_ln_rows(jnp.dot(m, a_ref[...], preferred_element_type=jnp.float32, precision=jax.lax.Precision.HIGHEST)
                            + e * b1_ref[...]), 0.2)
        t = _lrelu(_ln_rows(jnp.dot(t, w02_ref[...], preferred_element_type=jnp.float32, precision=jax.lax.Precision.HIGHEST)), 0.2)
        t = _ln_rows(jnp.dot(t, w03_ref[...], preferred_element_type=jnp.float32, precision=jax.lax.Precision.HIGHEST))
        idn = jnp.dot(m, p0m_ref[...], preferred_element_type=jnp.float32, precision=jax.lax.Precision.HIGHEST) + e * p0e_ref[...]
        h1 = _lrelu(t + idn, 0.2)

        t = _lrelu(_ln_rows(jnp.dot(h1, w11_ref[...], preferred_element_type=jnp.float32, precision=jax.lax.Precision.HIGHEST)), 0.2)
        t = _lrelu(_ln_rows(jnp.dot(t, w12_ref[...], preferred_element_type=jnp.float32, precision=jax.lax.Precision.HIGHEST)), 0.2)
        t = _ln_rows(jnp.dot(t, w13_ref[...], preferred_element_type=jnp.float32, precision=jax.lax.Precision.HIGHEST))
        h2 = _lrelu(t + h1, 0.2)

        t = _lrelu(_ln_rows(jnp.dot(h2, w21_ref[...], preferred_element_type=jnp.float32, precision=jax.lax.Precision.HIGHEST)), 0.2)
        t = _lrelu(_ln_rows(jnp.dot(t, w22_ref[...], preferred_element_type=jnp.float32, precision=jax.lax.Precision.HIGHEST)), 0.2)
        t = _ln_rows(jnp.dot(t, w23_ref[...], preferred_element_type=jnp.float32, precision=jax.lax.Precision.HIGHEST))
        h3 = _lrelu(t + jnp.dot(h2, p2m_ref[...], preferred_element_type=jnp.float32, precision=jax.lax.Precision.HIGHEST), 0.2)

        out_ref[...] = (jnp.dot(h3, fct_ref[...], preferred_element_type=jnp.float32, precision=jax.lax.Precision.HIGHEST)
                        + fcb_ref[...])

    return pl.pallas_call(
        body,
        out_shape=jax.ShapeDtypeStruct((b, out_dim), jnp.float32),
    )(p1, p2, env2, wm1, wm2, gm, bm, d0w1a, d0w1b, d0w2t, d0w3t, p0m, p0e,
      d1w1t, d1w2t, d1w3t, d2w1t, d2w2t, d2w3t, p2map, fct, fcb)


def _prep_layer_weights(aw1, aw2, uw, rw, cin, rm):
    eff = cin - 3 if rm else cin
    w1 = jnp.transpose(aw1[:, :eff])
    w2 = jnp.transpose(aw1[:, eff:2 * eff])
    w3 = jnp.transpose(aw1[:, 2 * eff:])
    uwt = jnp.transpose(uw)
    rwt = jnp.transpose(rw) if rw is not None else None
    if rm:
        pad = jnp.zeros((3, 64), jnp.float32)
        w1 = jnp.concatenate([pad, w1], axis=0)
        w2 = jnp.concatenate([pad, w2], axis=0)
        padc = jnp.zeros((3, uwt.shape[1]), jnp.float32)
        uwt = jnp.concatenate([padc, uwt], axis=0)
        if rwt is not None:
            rwt = jnp.concatenate([jnp.zeros((3, rwt.shape[1]), jnp.float32),
                                   rwt], axis=0)
    cout = uwt.shape[1]
    w23u = jnp.concatenate(
        [jnp.concatenate([w2, uwt], axis=1),
         jnp.concatenate([w3, jnp.zeros((16, cout), jnp.float32)], axis=1)],
        axis=0)
    return w1, w23u, aw2, rwt


def kernel(x, env, idx_base, mc0_aw1, mc0_aw2, mc0_uw, mc0_bg, mc0_bb, mc0_rw,
           mc1_aw1, mc1_aw2, mc1_uw, mc1_bg, mc1_bb,
           mc2_aw1, mc2_aw2, mc2_uw, mc2_bg, mc2_bb, mc2_rw,
           mc3_aw1, mc3_aw2, mc3_uw, mc3_bg, mc3_bb,
           conv_w, conv_bg, conv_bb, mrg_w, mrg_bg, mrg_bb,
           dec0_w1, dec0_w2, dec0_w3, dec1_w1, dec1_w2, dec1_w3,
           dec2_w1, dec2_w2, dec2_w3, fc_w, fc_b):
    b, cin0, n = x.shape
    emb = conv_w.shape[0]
    out_dim = fc_w.shape[0]
    tb = 2 if b % 2 == 0 else 1
    tb6 = 8 if b % 8 == 0 else 1
    bnk = b * n * _K

    xt = jnp.transpose(x, (0, 2, 1))

    w0 = _prep_layer_weights(mc0_aw1, mc0_aw2, mc0_uw, mc0_rw, cin0, True)
    w1 = _prep_layer_weights(mc1_aw1, mc1_aw2, mc1_uw, None, 32, False)
    w2 = _prep_layer_weights(mc2_aw1, mc2_aw2, mc2_uw, mc2_rw, 32, False)
    w3 = _prep_layer_weights(mc3_aw1, mc3_aw2, mc3_uw, None, 64, False)

    r2 = lambda v: v.reshape(1, -1)

    p0 = _layer_first_call(xt, w0, tb, n, 32)
    upd0, res0, s10, s20 = p0
    xc0, upd1, res1, s11, s21 = _layer_mid_call(
        (upd0, res0, s10, s20), r2(mc0_bg), r2(mc0_bb), w1, tb, n, 32,
        False, bnk)
    xc1, upd2, res2, s12, s22 = _layer_mid_call(
        (upd1, res1, s11, s21), r2(mc1_bg), r2(mc1_bb), w2, tb, n, 64,
        True, bnk)
    xc2, upd3, res3, s13, s23 = _layer_mid_call(
        (upd2, res2, s12, s22), r2(mc2_bg), r2(mc2_bb), w3, tb, n, 64,
        False, bnk)

    cw = jnp.transpose(conv_w)
    wcs = (cw[0:32], cw[32:64], cw[64:128], cw[128:192])
    y, ys1, ys2 = _conv_call((xc0, xc1, xc2), (upd3, res3, s13, s23),
                             r2(mc3_bg), r2(mc3_bb), wcs, tb, n, emb, bnk)

    p1, p2 = _pool_call(y, ys1, ys2, r2(conv_bg), r2(conv_bb), tb6, n, emb,
                        b * n)

    mrg_t = jnp.transpose(mrg_w)
    wm1, wm2 = mrg_t[:emb], mrg_t[emb:]
    d0w1t = jnp.transpose(dec0_w1)
    d0w1a, d0w1b = d0w1t[:emb], d0w1t[emb:]
    di0, do0 = dec0_w1.shape[1], dec0_w1.shape[0]
    p0full = (np.arange(di0)[:, None]
              == (np.arange(do0)[None, :] * di0) // do0).astype(np.float32)
    p0m = jnp.asarray(p0full[:emb])
    p0e = jnp.asarray(p0full[emb:])
    di2, do2 = dec2_w1.shape[1], dec2_w1.shape[0]
    p2map = jnp.asarray((np.arange(di2)[:, None]
                         == (np.arange(do2)[None, :] * di2) // do2)
                        .astype(np.float32))

    return _head_call(
        p1, p2, env.reshape(-1, 1), wm1, wm2, r2(mrg_bg), r2(mrg_bb),
        d0w1a, d0w1b, jnp.transpose(dec0_w2), jnp.transpose(dec0_w3),
        p0m, p0e,
        jnp.transpose(dec1_w1), jnp.transpose(dec1_w2), jnp.transpose(dec1_w3),
        jnp.transpose(dec2_w1), jnp.transpose(dec2_w2), jnp.transpose(dec2_w3),
        p2map,
        jnp.transpose(fc_w), fc_b.reshape(1, -1), out_dim)
